# baseline (speedup 1.0000x reference)
import functools

import jax
import jax.numpy as jnp
from jax import lax
from jax.experimental import pallas as pl
from jax.experimental.pallas import tpu as pltpu
from jax.experimental.pallas import tpu_sc as plsc

_DIM = 4096
_E = 64
_R = 8
_IN = 2 * _DIM
_BZ = 4
_SEQ = 2048
_ROWS = 512
_NSTEPS = _BZ * _SEQ // _ROWS
_NC = 2
_NS = 16
_L = 16
_NEG = -1e30

_sc_mesh = plsc.VectorSubcoreMesh(
    core_axis_name="c", subcore_axis_name="s", num_cores=_NC, num_subcores=_NS)


def _vsum(v):
    return plsc.cumsum(v)[_L - 1]


def _vmax(v):
    return plsc.cummax(v)[_L - 1]


def _vmin_i32(v):
    return -plsc.cummax(-v)[_L - 1]


@functools.partial(
    pl.kernel,
    out_type=jax.ShapeDtypeStruct((_NC * _BZ * _E,), jnp.float32),
    mesh=_sc_mesh,
    compiler_params=pltpu.CompilerParams(needs_layout_passes=False),
    scratch_types=[
        pltpu.VMEM((_IN,), jnp.float32),
        pltpu.VMEM((_IN,), jnp.float32),
        pltpu.VMEM((_L,), jnp.float32),
        pltpu.VMEM((_L,), jnp.float32),
        pltpu.VMEM((4 * _E,), jnp.float32),
        pltpu.VMEM((_NS * _L,), jnp.float32),
        pltpu.VMEM((_E,), jnp.float32),
        pltpu.VMEM_SHARED((4 * _E,), jnp.float32),
        pltpu.VMEM_SHARED((_NS * _L,), jnp.float32),
        pltpu.SemaphoreType.DMA,
        pltpu.SemaphoreType.DMA,
    ],
)
def _sc_gate2(wp_hbm, la_hbm, out_hbm, buf0, buf1, stage16, a16,
              wp_local, a_local, orow, wp_sh, a_sh, sem0, sem1):
    c = lax.axis_index("c")
    t = lax.axis_index("s")
    iota = lax.iota(jnp.int32, _L)
    bufs = (buf0, buf1)
    sems = (sem0, sem1)

    def src_wp(p):
        e = 4 * t + p // 4
        r = 4 * c + (p % 4)
        return wp_hbm.at[pl.ds(e * (_R * _IN) + r * _IN, _IN)]

    def src_la():
        b = t // 4
        r = 4 * c + (t % 4)
        return la_hbm.at[pl.ds((b * _R + r) * _IN, _IN)]

    def reduce_buf(buf):
        zero = jnp.zeros((_L,), jnp.float32)

        def step(i, accs):
            a0, a1, a2, a3 = accs
            base = i * 64
            return (a0 + buf[pl.ds(base, _L)],
                    a1 + buf[pl.ds(base + 16, _L)],
                    a2 + buf[pl.ds(base + 32, _L)],
                    a3 + buf[pl.ds(base + 48, _L)])

        a0, a1, a2, a3 = lax.fori_loop(0, _IN // 64, step, (zero,) * 4)
        return _vsum(a0 + a1 + a2 + a3)

    cp = pltpu.async_copy(src_wp(0), buf0, sem0)
    stage = jnp.zeros((_L,), jnp.float32)
    for p in range(16):
        cur = bufs[p % 2]
        nxt = bufs[(p + 1) % 2]
        nsem = sems[(p + 1) % 2]
        ncp = pltpu.async_copy(src_wp(p + 1) if p < 15 else src_la(),
                               nxt, nsem)
        cp.wait()
        stage = jnp.where(iota == p, reduce_buf(cur), stage)
        cp = ncp
    stage16[...] = stage
    pltpu.sync_copy(stage16, wp_sh.at[pl.ds(_L * t, _L)])
    cp.wait()
    a16[...] = jnp.full((_L,), reduce_buf(bufs[0]), jnp.float32)
    pltpu.sync_copy(a16, a_sh.at[pl.ds(_L * t, _L)])
    plsc.subcore_barrier()

    @pl.when(t < _BZ)
    def _():
        b = t
        pltpu.sync_copy(wp_sh, wp_local)
        pltpu.sync_copy(a_sh, a_local)
        acc = [jnp.zeros((_L,), jnp.float32) for _ in range(4)]
        for rl in range(4):
            a = a_local[pl.ds((b * 4 + rl) * _L, _L)][0]
            l2 = [a * plsc.load_gather(wp_local, [(iota + _L * k) * 4 + rl])
                  for k in range(4)]
            m = _vmax(jnp.maximum(jnp.maximum(l2[0], l2[1]),
                                  jnp.maximum(l2[2], l2[3])))
            ex = [jnp.exp(v - m) for v in l2]
            tot = _vsum(ex[0] + ex[1] + ex[2] + ex[3])
            acc = [acc[k] + ex[k] / tot for k in range(4)]
        for k in range(4):
            orow[pl.ds(_L * k, _L)] = acc[k]
        pltpu.sync_copy(orow, out_hbm.at[pl.ds((c * _BZ + b) * _E, _E)])


@functools.partial(
    pl.kernel,
    out_type=jax.ShapeDtypeStruct((_BZ * _L,), jnp.int32),
    mesh=_sc_mesh,
    compiler_params=pltpu.CompilerParams(needs_layout_passes=False),
    scratch_types=[
        pltpu.VMEM((_E,), jnp.float32),
        pltpu.VMEM((_E,), jnp.float32),
        pltpu.VMEM((_E,), jnp.float32),
        pltpu.VMEM((_L,), jnp.int32),
    ],
)
def _sc_topk(s1_hbm, s2_hbm, out_hbm, v1b, h0b, h1b, ob):
    c = lax.axis_index("c")
    t = lax.axis_index("s")
    iota = lax.iota(jnp.int32, _L)

    @pl.when((c == 0) & (t < _BZ))
    def _():
        b = t
        pltpu.sync_copy(s1_hbm.at[pl.ds(b * _E, _E)], v1b)
        pltpu.sync_copy(s2_hbm.at[pl.ds(b * _E, _E)], h0b)
        pltpu.sync_copy(s2_hbm.at[pl.ds((_BZ + b) * _E, _E)], h1b)
        s2 = [h0b[pl.ds(_L * k, _L)] + h1b[pl.ds(_L * k, _L)]
              for k in range(4)]
        m = _vmax(jnp.maximum(jnp.maximum(s2[0], s2[1]),
                              jnp.maximum(s2[2], s2[3])))
        ex = [jnp.exp(v - m) for v in s2]
        tot = _vsum(ex[0] + ex[1] + ex[2] + ex[3])
        vals = [v1b[pl.ds(_L * k, _L)] + ex[k] / tot for k in range(4)]
        iotas = [iota + _L * k for k in range(4)]
        outv = jnp.zeros((_L,), jnp.int32)
        for kk in range(_R):
            m2 = _vmax(jnp.maximum(jnp.maximum(vals[0], vals[1]),
                                   jnp.maximum(vals[2], vals[3])))
            cand = [jnp.where(vals[k] >= m2, iotas[k], _E) for k in range(4)]
            idx = _vmin_i32(jnp.minimum(jnp.minimum(cand[0], cand[1]),
                                        jnp.minimum(cand[2], cand[3])))
            outv = jnp.where(iota == kk, idx, outv)
            vals = [jnp.where(iotas[k] == idx, _NEG, vals[k])
                    for k in range(4)]
        ob[...] = outv
        pltpu.sync_copy(ob, out_hbm.at[pl.ds(b * _L, _L)])


def _softmax_lanes(x):
    m = jnp.max(x, axis=-1, keepdims=True)
    e = jnp.exp(x - m)
    return e / jnp.sum(e, axis=-1, keepdims=True)


def _dot_t(a, b):
    return jax.lax.dot_general(
        a, b, (((1,), (1,)), ((), ())), preferred_element_type=jnp.float32)


def _tc_body(x_ref, wb_ref, instr_ref, out_ref, acc_ref):
    s = pl.program_id(0)

    @pl.when(s == 0)
    def _():
        acc_ref[...] = jnp.zeros_like(acc_ref)

    blk = x_ref[...]
    ones = jnp.ones((1, _ROWS), jnp.float32)
    partial = jnp.dot(ones, blk, preferred_element_type=jnp.float32)
    b = s // (_SEQ // _ROWS)
    acc_ref[pl.ds(b, 1), :] += partial

    @pl.when(s == _NSTEPS - 1)
    def _():
        mean = acc_ref[...] * (1.0 / _SEQ)
        wb = wb_ref[...]
        l1 = _dot_t(instr_ref[...], wb[:, :_DIM]) + _dot_t(mean, wb[:, _DIM:])
        out_ref[...] = _softmax_lanes(l1)


@jax.jit
def kernel(input_x, instr_x, lora_A_param, W_B, W_P):
    s2h = _sc_gate2(W_P.reshape(-1), lora_A_param.reshape(-1))
    s1 = pl.pallas_call(
        _tc_body,
        grid=(_NSTEPS,),
        in_specs=[
            pl.BlockSpec((_ROWS, _DIM), lambda s: (s, 0)),
            pl.BlockSpec((_E, _IN), lambda s: (0, 0)),
            pl.BlockSpec((_BZ, _DIM), lambda s: (0, 0)),
        ],
        out_specs=pl.BlockSpec((_BZ, _E), lambda s: (0, 0)),
        out_shape=jax.ShapeDtypeStruct((_BZ, _E), jnp.float32),
        scratch_shapes=[pltpu.VMEM((_BZ, _DIM), jnp.float32)],
    )(input_x.reshape(_BZ * _SEQ, _DIM), W_B, instr_x)
    idx = _sc_topk(s1.reshape(-1), s2h)
    return idx.reshape(_BZ, _L)[:, :_R]


# speedup vs baseline: 1.2678x; 1.2678x over previous
import functools

import jax
import jax.numpy as jnp
from jax import lax
from jax.experimental import pallas as pl
from jax.experimental.pallas import tpu as pltpu
from jax.experimental.pallas import tpu_sc as plsc

_DIM = 4096
_E = 64
_R = 8
_IN = 2 * _DIM
_BZ = 4
_SEQ = 2048
_ROWS = 512
_NSTEPS = _BZ * _SEQ // _ROWS
_NC = 2
_NS = 16
_L = 16
_NEG = -1e30

_sc_mesh = plsc.VectorSubcoreMesh(
    core_axis_name="c", subcore_axis_name="s", num_cores=_NC, num_subcores=_NS)


def _vsum(v):
    return plsc.cumsum(v)[_L - 1]


def _vmax(v):
    return plsc.cummax(v)[_L - 1]


def _vmin_i32(v):
    return -plsc.cummax(-v)[_L - 1]


@functools.partial(
    pl.kernel,
    out_type=jax.ShapeDtypeStruct((_NC * _BZ * _E,), jnp.float32),
    mesh=_sc_mesh,
    compiler_params=pltpu.CompilerParams(needs_layout_passes=False),
    scratch_types=[
        pltpu.VMEM((_IN,), jnp.float32),
        pltpu.VMEM((_IN,), jnp.float32),
        pltpu.VMEM((_L,), jnp.float32),
        pltpu.VMEM((_L,), jnp.float32),
        pltpu.VMEM((4 * _E,), jnp.float32),
        pltpu.VMEM((_NS * _L,), jnp.float32),
        pltpu.VMEM((_E,), jnp.float32),
        pltpu.VMEM_SHARED((4 * _E,), jnp.float32),
        pltpu.VMEM_SHARED((_NS * _L,), jnp.float32),
        pltpu.SemaphoreType.DMA,
        pltpu.SemaphoreType.DMA,
    ],
)
def _sc_gate2(wp_hbm, la_hbm, out_hbm, buf0, buf1, stage16, a16,
              wp_local, a_local, orow, wp_sh, a_sh, sem0, sem1):
    c = lax.axis_index("c")
    t = lax.axis_index("s")
    iota = lax.iota(jnp.int32, _L)
    bufs = (buf0, buf1)
    sems = (sem0, sem1)

    def src_wp(p):
        e = 4 * t + p // 4
        r = 4 * c + (p % 4)
        return wp_hbm.at[e, pl.ds(r * _IN, _IN)]

    def src_la():
        b = t // 4
        r = 4 * c + (t % 4)
        return la_hbm.at[b, r, :]

    def reduce_buf(buf):
        zero = jnp.zeros((_L,), jnp.float32)

        def step(i, accs):
            a0, a1, a2, a3 = accs
            base = i * 256
            for j in range(4):
                a0 = a0 + buf[pl.ds(base + j * 64, _L)]
                a1 = a1 + buf[pl.ds(base + j * 64 + 16, _L)]
                a2 = a2 + buf[pl.ds(base + j * 64 + 32, _L)]
                a3 = a3 + buf[pl.ds(base + j * 64 + 48, _L)]
            return (a0, a1, a2, a3)

        a0, a1, a2, a3 = lax.fori_loop(0, _IN // 256, step, (zero,) * 4)
        return _vsum(a0 + a1 + a2 + a3)

    cp = pltpu.async_copy(src_wp(0), buf0, sem0)
    stage = jnp.zeros((_L,), jnp.float32)
    for p in range(16):
        cur = bufs[p % 2]
        nxt = bufs[(p + 1) % 2]
        nsem = sems[(p + 1) % 2]
        ncp = pltpu.async_copy(src_wp(p + 1) if p < 15 else src_la(),
                               nxt, nsem)
        cp.wait()
        stage = jnp.where(iota == p, reduce_buf(cur), stage)
        cp = ncp
    stage16[...] = stage
    pltpu.sync_copy(stage16, wp_sh.at[pl.ds(_L * t, _L)])
    cp.wait()
    a16[...] = jnp.full((_L,), reduce_buf(bufs[0]), jnp.float32)
    pltpu.sync_copy(a16, a_sh.at[pl.ds(_L * t, _L)])
    plsc.subcore_barrier()

    @pl.when(t < _BZ)
    def _():
        b = t
        pltpu.sync_copy(wp_sh, wp_local)
        pltpu.sync_copy(a_sh, a_local)
        acc = [jnp.zeros((_L,), jnp.float32) for _ in range(4)]
        for rl in range(4):
            a = a_local[pl.ds((b * 4 + rl) * _L, _L)][0]
            l2 = [a * plsc.load_gather(wp_local, [(iota + _L * k) * 4 + rl])
                  for k in range(4)]
            m = _vmax(jnp.maximum(jnp.maximum(l2[0], l2[1]),
                                  jnp.maximum(l2[2], l2[3])))
            ex = [jnp.exp(v - m) for v in l2]
            tot = _vsum(ex[0] + ex[1] + ex[2] + ex[3])
            acc = [acc[k] + ex[k] / tot for k in range(4)]
        for k in range(4):
            orow[pl.ds(_L * k, _L)] = acc[k]
        pltpu.sync_copy(orow, out_hbm.at[pl.ds((c * _BZ + b) * _E, _E)])


@functools.partial(
    pl.kernel,
    out_type=jax.ShapeDtypeStruct((_BZ * _L,), jnp.int32),
    mesh=_sc_mesh,
    compiler_params=pltpu.CompilerParams(needs_layout_passes=False),
    scratch_types=[
        pltpu.VMEM((_E,), jnp.float32),
        pltpu.VMEM((_E,), jnp.float32),
        pltpu.VMEM((_E,), jnp.float32),
        pltpu.VMEM((_L,), jnp.int32),
    ],
)
def _sc_topk(s1_hbm, s2_hbm, out_hbm, v1b, h0b, h1b, ob):
    c = lax.axis_index("c")
    t = lax.axis_index("s")
    iota = lax.iota(jnp.int32, _L)

    @pl.when((c == 0) & (t < _BZ))
    def _():
        b = t
        pltpu.sync_copy(s1_hbm.at[b, :], v1b)
        pltpu.sync_copy(s2_hbm.at[pl.ds(b * _E, _E)], h0b)
        pltpu.sync_copy(s2_hbm.at[pl.ds((_BZ + b) * _E, _E)], h1b)
        s2 = [h0b[pl.ds(_L * k, _L)] + h1b[pl.ds(_L * k, _L)]
              for k in range(4)]
        m = _vmax(jnp.maximum(jnp.maximum(s2[0], s2[1]),
                              jnp.maximum(s2[2], s2[3])))
        ex = [jnp.exp(v - m) for v in s2]
        tot = _vsum(ex[0] + ex[1] + ex[2] + ex[3])
        vals = [v1b[pl.ds(_L * k, _L)] + ex[k] / tot for k in range(4)]
        iotas = [iota + _L * k for k in range(4)]
        outv = jnp.zeros((_L,), jnp.int32)
        for kk in range(_R):
            m2 = _vmax(jnp.maximum(jnp.maximum(vals[0], vals[1]),
                                   jnp.maximum(vals[2], vals[3])))
            cand = [jnp.where(vals[k] >= m2, iotas[k], _E) for k in range(4)]
            idx = _vmin_i32(jnp.minimum(jnp.minimum(cand[0], cand[1]),
                                        jnp.minimum(cand[2], cand[3])))
            outv = jnp.where(iota == kk, idx, outv)
            vals = [jnp.where(iotas[k] == idx, _NEG, vals[k])
                    for k in range(4)]
        ob[...] = outv
        pltpu.sync_copy(ob, out_hbm.at[pl.ds(b * _L, _L)])


def _softmax_lanes(x):
    m = jnp.max(x, axis=-1, keepdims=True)
    e = jnp.exp(x - m)
    return e / jnp.sum(e, axis=-1, keepdims=True)


def _dot_t(a, b):
    return jax.lax.dot_general(
        a, b, (((1,), (1,)), ((), ())), preferred_element_type=jnp.float32)


def _tc_body(x_ref, wb_ref, instr_ref, out_ref, acc_ref):
    s = pl.program_id(0)

    @pl.when(s == 0)
    def _():
        acc_ref[...] = jnp.zeros_like(acc_ref)

    blk = x_ref[...]
    ones = jnp.ones((1, _ROWS), jnp.float32)
    partial = jnp.dot(ones, blk, preferred_element_type=jnp.float32)
    b = s // (_SEQ // _ROWS)
    acc_ref[pl.ds(b, 1), :] += partial

    @pl.when(s == _NSTEPS - 1)
    def _():
        mean = acc_ref[...] * (1.0 / _SEQ)
        wb = wb_ref[...]
        l1 = _dot_t(instr_ref[...], wb[:, :_DIM]) + _dot_t(mean, wb[:, _DIM:])
        out_ref[...] = _softmax_lanes(l1)


@jax.jit
def kernel(input_x, instr_x, lora_A_param, W_B, W_P):
    s2h = _sc_gate2(W_P, lora_A_param)
    s1 = pl.pallas_call(
        _tc_body,
        grid=(_NSTEPS,),
        in_specs=[
            pl.BlockSpec((_ROWS, _DIM), lambda s: (s, 0)),
            pl.BlockSpec((_E, _IN), lambda s: (0, 0)),
            pl.BlockSpec((_BZ, _DIM), lambda s: (0, 0)),
        ],
        out_specs=pl.BlockSpec((_BZ, _E), lambda s: (0, 0)),
        out_shape=jax.ShapeDtypeStruct((_BZ, _E), jnp.float32),
        scratch_shapes=[pltpu.VMEM((_BZ, _DIM), jnp.float32)],
    )(input_x.reshape(_BZ * _SEQ, _DIM), W_B, instr_x)
    idx = _sc_topk(s1, s2h)
    return idx.reshape(_BZ, _L)[:, :_R]


# speedup vs baseline: 1.7210x; 1.3574x over previous
import functools

import jax
import jax.numpy as jnp
from jax.experimental import pallas as pl
from jax.experimental.pallas import tpu as pltpu

_DIM = 4096
_E = 64
_R = 8
_IN = 2 * _DIM
_BZ = 4
_SEQ = 2048
_SCHUNK = 256
_NSTEPS = _SEQ // _SCHUNK
_WPCHUNK = _IN


def _softmax_lanes(x):
    m = jnp.max(x, axis=-1, keepdims=True)
    e = jnp.exp(x - m)
    return e / jnp.sum(e, axis=-1, keepdims=True)


def _dot_t(a, b):
    return jax.lax.dot_general(
        a, b, (((1,), (1,)), ((), ())), preferred_element_type=jnp.float32)


def _body(x_ref, wp_ref, wb_ref, instr_ref, la_ref, out_ref, acc_ref, wps_ref):
    s = pl.program_id(0)

    @pl.when(s == 0)
    def _():
        acc_ref[...] = jnp.zeros_like(acc_ref)

    blk = x_ref[...]
    for b in range(_BZ):
        acc_ref[b : b + 1, :] += jnp.sum(blk[b], axis=0, keepdims=True)

    @pl.when(s < _R)
    def _():
        ones = jnp.ones((1, _WPCHUNK), jnp.float32)
        row = _dot_t(ones, wp_ref[...])
        wps_ref[pl.ds(s, 1), :] = row

    @pl.when(s == _NSTEPS - 1)
    def _():
        mean = acc_ref[...] * (1.0 / _SEQ)
        wb = wb_ref[...]
        l1 = _dot_t(instr_ref[...], wb[:, :_DIM]) + _dot_t(mean, wb[:, _DIM:])
        s1 = _softmax_lanes(l1)

        s2 = jnp.zeros((_BZ, _E), jnp.float32)
        for r in range(_R):
            a_r = jnp.sum(la_ref[:, r, :], axis=-1, keepdims=True)
            l2r = a_r * wps_ref[r : r + 1, :]
            s2 = s2 + _softmax_lanes(l2r)
        logits = s1 + _softmax_lanes(s2)

        iota = jax.lax.broadcasted_iota(jnp.int32, (_BZ, _E), 1)
        iota8 = jax.lax.broadcasted_iota(jnp.int32, (_BZ, _R), 1)
        vals = logits
        out = jnp.zeros((_BZ, _R), jnp.int32)
        for k in range(_R):
            m = jnp.max(vals, axis=-1, keepdims=True)
            idx = jnp.min(jnp.where(vals >= m, iota, _E), axis=-1,
                          keepdims=True)
            out = jnp.where(iota8 == k, idx, out)
            vals = jnp.where(iota == idx, -jnp.inf, vals)
        out_ref[...] = out


@jax.jit
def kernel(input_x, instr_x, lora_A_param, W_B, W_P):
    return pl.pallas_call(
        _body,
        grid=(_NSTEPS,),
        in_specs=[
            pl.BlockSpec((_BZ, _SCHUNK, _DIM), lambda s: (0, s, 0)),
            pl.BlockSpec((_E, _WPCHUNK), lambda s: (0, jnp.minimum(s, _R - 1))),
            pl.BlockSpec((_E, _IN), lambda s: (0, 0)),
            pl.BlockSpec((_BZ, _DIM), lambda s: (0, 0)),
            pl.BlockSpec((_BZ, _R, _IN), lambda s: (0, 0, 0)),
        ],
        out_specs=pl.BlockSpec((_BZ, _R), lambda s: (0, 0)),
        out_shape=jax.ShapeDtypeStruct((_BZ, _R), jnp.int32),
        scratch_shapes=[
            pltpu.VMEM((_BZ, _DIM), jnp.float32),
            pltpu.VMEM((_R, _E), jnp.float32),
        ],
    )(input_x, W_P, W_B, instr_x, lora_A_param)
